# baseline (speedup 1.0000x reference)
import functools

import jax
import jax.numpy as jnp
from jax import lax
from jax.experimental import pallas as pl
from jax.experimental.pallas import tpu as pltpu
from jax.experimental.pallas import tpu_sc as plsc

_RB = 16
_VT = 1024


def _sc_part(logs, out_ref, V_SC, B, V):
    info = plsc.get_sparse_core_info()
    NC, NS, L = info.num_cores, info.num_subcores, info.num_lanes
    NW = NC * NS
    CH = V_SC // NW
    npair = CH // (2 * _RB)

    mesh = plsc.VectorSubcoreMesh(core_axis_name="c", subcore_axis_name="s")

    @functools.partial(
        pl.kernel,
        mesh=mesh,
        scratch_types=[
            pltpu.VMEM((CH,), jnp.float32),
            pltpu.VMEM((_RB, B), jnp.float32),
            pltpu.VMEM((_RB, B), jnp.float32),
            pltpu.SemaphoreType.DMA,
            pltpu.SemaphoreType.DMA,
        ],
        compiler_params=pltpu.CompilerParams(needs_layout_passes=False),
    )
    def bcast_sc(logs_hbm, out_hbm, lg_v, buf0, buf1, sem0, sem1):
        wid = lax.axis_index("s") * NC + lax.axis_index("c")
        base = wid * CH
        pltpu.sync_copy(logs_hbm.at[pl.ds(base, CH)], lg_v)
        bufs = (buf0, buf1)
        sems = (sem0, sem1)
        col_idx = [lax.iota(jnp.int32, L) + j * L for j in range(B // L)]

        def fill(buf, ch):
            def row_body(r, carry):
                idx = jnp.full((L,), ch * _RB + r, jnp.int32)
                vec = plsc.load_gather(lg_v, [idx])
                row_idx = jnp.full((L,), r, jnp.int32)
                for j in range(B // L):
                    plsc.store_scatter(buf, [row_idx, col_idx[j]], vec)
                return carry

            lax.fori_loop(0, _RB, row_body, 0)

        def start(p, ch):
            pltpu.make_async_copy(
                bufs[p], out_hbm.at[pl.ds(base + ch * _RB, _RB)], sems[p]
            ).start()

        def drain(p, ch):
            pltpu.make_async_copy(
                bufs[p], out_hbm.at[pl.ds(base + ch * _RB, _RB)], sems[p]
            ).wait()

        for p in range(2):
            fill(bufs[p], p)
            start(p, p)

        def step(k2, carry):
            for p in range(2):
                ch = k2 * 2 + p
                drain(p, ch)
                fill(bufs[p], ch)
                start(p, ch)
            return carry

        lax.fori_loop(1, npair, step, 0)
        for p in range(2):
            drain(p, p)

    bcast_sc(logs, out_ref)


def _tc_part(logs_col, out_ref, V_SC, B, V):
    nch = -(-(V - V_SC) // _VT)
    mesh = pltpu.create_tensorcore_mesh("tc")
    NCT = int(mesh.devices.size)
    ncpc = nch // NCT

    @functools.partial(
        pl.kernel,
        mesh=mesh,
        scratch_types=[
            pltpu.VMEM((_VT, B), jnp.float32),
            pltpu.VMEM((_VT, B), jnp.float32),
            pltpu.VMEM((_VT, 1), jnp.float32),
            pltpu.VMEM((_VT, 1), jnp.float32),
            pltpu.SemaphoreType.DMA,
            pltpu.SemaphoreType.DMA,
            pltpu.SemaphoreType.DMA,
            pltpu.SemaphoreType.DMA,
        ],
    )
    def bcast_tc(lg_hbm, out_hbm, buf0, buf1, lg0, lg1, s0, s1, ls0, ls1):
        bufs = (buf0, buf1)
        lgs = (lg0, lg1)
        sems = (s0, s1)
        lsems = (ls0, ls1)
        cid = lax.axis_index("tc")

        def row0_of(k):
            return jnp.minimum(V_SC + (k * NCT + cid) * _VT, V - _VT)

        def lg_start(p, ch):
            pltpu.make_async_copy(
                lg_hbm.at[pl.ds(row0_of(ch), _VT)], lgs[p], lsems[p]
            ).start()

        def lg_wait(p, ch):
            pltpu.make_async_copy(
                lg_hbm.at[pl.ds(row0_of(ch), _VT)], lgs[p], lsems[p]
            ).wait()

        def out_start(p, ch):
            pltpu.make_async_copy(
                bufs[p], out_hbm.at[pl.ds(row0_of(ch), _VT)], sems[p]
            ).start()

        def out_wait(p, ch):
            pltpu.make_async_copy(
                bufs[p], out_hbm.at[pl.ds(row0_of(ch), _VT)], sems[p]
            ).wait()

        for p in range(2):
            lg_start(p, p)
        for p in range(2):
            lg_wait(p, p)
            bufs[p][...] = jnp.broadcast_to(lgs[p][...], (_VT, B))
            out_start(p, p)

        def step(k2, carry):
            for p in range(2):
                ch = k2 * 2 + p
                lg_start(p, ch)
                out_wait(p, ch)
                lg_wait(p, ch)
                bufs[p][...] = jnp.broadcast_to(lgs[p][...], (_VT, B))
                out_start(p, ch)
            return carry

        lax.fori_loop(1, ncpc // 2, step, 0)
        for p in range(2):
            out_wait(p, p)

    bcast_tc(logs_col, out_ref)


def kernel(hist, idx, logs):
    B = hist.shape[1]
    V = logs.shape[0] - 1
    V_SC = 54 * 32 * 2 * _RB

    out_ref = jax.new_ref(jax.lax.empty((V, B), jnp.float32))
    logs_col = logs[:V].reshape(V, 1)
    _sc_part(logs, out_ref, V_SC, B, V)
    _tc_part(logs_col, out_ref, V_SC, B, V)
    return out_ref[...].T


# speedup vs baseline: 1.1875x; 1.1875x over previous
import functools

import jax
import jax.numpy as jnp
from jax import lax
from jax.experimental import pallas as pl
from jax.experimental.pallas import tpu as pltpu
from jax.experimental.pallas import tpu_sc as plsc

_RB = 16


def kernel(hist, idx, logs):
    B = hist.shape[1]
    V = logs.shape[0] - 1

    info = plsc.get_sparse_core_info()
    NC, NS, L = info.num_cores, info.num_subcores, info.num_lanes
    NW = NC * NS
    CH = -(-V // NW)
    CH = -(-CH // (2 * _RB)) * (2 * _RB)
    npair = CH // (2 * _RB)

    mesh = plsc.VectorSubcoreMesh(core_axis_name="c", subcore_axis_name="s")

    @functools.partial(
        pl.kernel,
        mesh=mesh,
        out_type=jax.ShapeDtypeStruct((V, B), jnp.float32),
        scratch_types=[
            pltpu.VMEM((CH,), jnp.float32),
            pltpu.VMEM((_RB, B), jnp.float32),
            pltpu.VMEM((_RB, B), jnp.float32),
            pltpu.SemaphoreType.DMA,
            pltpu.SemaphoreType.DMA,
        ],
        compiler_params=pltpu.CompilerParams(needs_layout_passes=False),
    )
    def bcast_t(logs_hbm, out_hbm, lg_v, buf0, buf1, sem0, sem1):
        wid = lax.axis_index("s") * NC + lax.axis_index("c")
        base = jnp.minimum(wid * CH, V - CH)
        pltpu.sync_copy(logs_hbm.at[pl.ds(base, CH)], lg_v)
        bufs = (buf0, buf1)
        sems = (sem0, sem1)

        col_idx = [lax.iota(jnp.int32, L) + j * L for j in range(B // L)]

        def fill(buf, ch):
            def row_body(r, carry):
                idx = jnp.full((L,), ch * _RB + r, jnp.int32)
                vec = plsc.load_gather(lg_v, [idx])
                row_idx = jnp.full((L,), r, jnp.int32)
                for j in range(B // L):
                    plsc.store_scatter(buf, [row_idx, col_idx[j]], vec)
                return carry

            lax.fori_loop(0, _RB, row_body, 0)

        def start(p, ch):
            cp = pltpu.make_async_copy(
                bufs[p], out_hbm.at[pl.ds(base + ch * _RB, _RB)], sems[p]
            )
            cp.start()

        def drain(p, ch):
            pltpu.make_async_copy(
                bufs[p], out_hbm.at[pl.ds(base + ch * _RB, _RB)], sems[p]
            ).wait()

        for p in range(2):
            fill(bufs[p], p)
            start(p, p)

        def step(k2, carry):
            for p in range(2):
                ch = k2 * 2 + p
                drain(p, ch)
                fill(bufs[p], ch)
                start(p, ch)
            return carry

        lax.fori_loop(1, npair, step, 0)
        for p in range(2):
            drain(p, p)

    out_t = bcast_t(logs)
    return out_t.T
